# baseline (speedup 1.0000x reference)
import functools

import jax
import jax.numpy as jnp
from jax import lax
from jax.experimental import pallas as pl
from jax.experimental.pallas import tpu as pltpu
from jax.experimental.pallas import tpu_sc as plsc

D = 128
_NC, _NS = 2, 16
_NW = _NC * _NS
_L = 16


def _pick_block(n, candidates=(1280, 1024, 1000, 800, 512, 400, 256, 200, 160, 128, 80, 40, 16, 8)):
    for c in candidates:
        if n % c == 0:
            return c
    return n


def _pack32(x16):
    n, d = x16.shape
    return jax.lax.bitcast_convert_type(x16.reshape(n, d // 2, 2), jnp.int32)


def _unpack16(x32):
    n, h = x32.shape
    return jax.lax.bitcast_convert_type(x32, jnp.bfloat16).reshape(n, 2 * h)


def _ln(x, gamma, beta, eps=1e-5):
    mu = jnp.mean(x, axis=-1, keepdims=True)
    xc = x - mu
    var = jnp.mean(xc * xc, axis=-1, keepdims=True)
    return xc * jax.lax.rsqrt(var + eps) * gamma + beta


def _resmlp_body(x_ref, w_ref, g_ref, b_ref, out_ref):
    x = x_ref[...]
    h = jnp.tanh(jnp.dot(x, w_ref[...], preferred_element_type=jnp.float32))
    out_ref[...] = x + _ln(h, g_ref[...], b_ref[...])


def _proj_body(x_ref, w_ref, out_ref):
    out_ref[...] = jnp.dot(
        x_ref[...], w_ref[...], preferred_element_type=jnp.float32
    ).astype(out_ref.dtype)


def _edge_body(bond_ref, gr_ref, gm_ref, wb_ref, g_ref, b_ref, de_ref, ob_ref):
    x = bond_ref[...]
    h = jnp.tanh(jnp.dot(x, wb_ref[...], preferred_element_type=jnp.float32)
                 + gr_ref[...].astype(jnp.float32)
                 + gm_ref[...].astype(jnp.float32))
    d = _ln(h, g_ref[...], b_ref[...])
    de_ref[...] = d
    ob_ref[...] = x + d


def _meshout_body(mesh_ref, agg_ref, w1_ref, w2_ref, g_ref, b_ref, out_ref):
    x = mesh_ref[...]
    h = jnp.tanh(jnp.dot(x, w1_ref[...], preferred_element_type=jnp.float32)
                 + jnp.dot(agg_ref[...], w2_ref[...], preferred_element_type=jnp.float32))
    out_ref[...] = x + _ln(h, g_ref[...], b_ref[...])


def _row_spec(tile):
    return pl.BlockSpec((tile, D), lambda i: (i, 0))


def _const_spec(shape):
    return pl.BlockSpec(shape, lambda i: (0,) * len(shape))


def _resmlp_stage(x, w, g, b):
    n = x.shape[0]
    t = _pick_block(n)
    return pl.pallas_call(
        _resmlp_body,
        grid=(n // t,),
        in_specs=[_row_spec(t), _const_spec((D, D)),
                  _const_spec((1, D)), _const_spec((1, D))],
        out_specs=_row_spec(t),
        out_shape=jax.ShapeDtypeStruct((n, D), jnp.float32),
    )(x, w, g, b)


def _proj_stage(x, w, out_dtype=jnp.float32):
    n = x.shape[0]
    t = _pick_block(n)
    return pl.pallas_call(
        _proj_body,
        grid=(n // t,),
        in_specs=[_row_spec(t), _const_spec((D, D))],
        out_specs=_row_spec(t),
        out_shape=jax.ShapeDtypeStruct((n, D), out_dtype),
    )(x, w)


def _edge_stage(bond, gr, gm, wb, g, b):
    n = bond.shape[0]
    t = _pick_block(n)
    return pl.pallas_call(
        _edge_body,
        grid=(n // t,),
        in_specs=[_row_spec(t), _row_spec(t), _row_spec(t), _const_spec((D, D)),
                  _const_spec((1, D)), _const_spec((1, D))],
        out_specs=[_row_spec(t), _row_spec(t)],
        out_shape=[jax.ShapeDtypeStruct((n, D), jnp.float32),
                   jax.ShapeDtypeStruct((n, D), jnp.float32)],
    )(bond, gr, gm, wb, g, b)


def _meshout_stage(mesh, agg, w1, w2, g, b):
    n = mesh.shape[0]
    t = _pick_block(n)
    return pl.pallas_call(
        _meshout_body,
        grid=(n // t,),
        in_specs=[_row_spec(t), _row_spec(t), _const_spec((D, D)),
                  _const_spec((D, D)), _const_spec((1, D)), _const_spec((1, D))],
        out_specs=_row_spec(t),
        out_shape=jax.ShapeDtypeStruct((n, D), jnp.float32),
    )(mesh, agg, w1, w2, g, b)


def _sc_mesh():
    return plsc.VectorSubcoreMesh(core_axis_name="c", subcore_axis_name="s",
                                  num_cores=_NC, num_subcores=_NS)


def _edge_gather(pr, pm, src, dst):
    e = src.shape[0]
    ech = 128
    e_pad = -(-e // (_NW * 2 * ech)) * (_NW * 2 * ech)
    epw = e_pad // _NW
    nch = epw // ech
    src3 = jnp.pad(src, (0, e_pad - e)).reshape(_NW, nch, ech)
    dst3 = jnp.pad(dst, (0, e_pad - e)).reshape(_NW, nch, ech)
    dw = pr.shape[1]

    @functools.partial(
        pl.kernel,
        out_type=[jax.ShapeDtypeStruct((e_pad, dw), jnp.int32),
                  jax.ShapeDtypeStruct((e_pad, dw), jnp.int32)],
        mesh=_sc_mesh(),
        scratch_types=[
            pltpu.VMEM((nch, ech), jnp.int32),
            pltpu.VMEM((nch, ech), jnp.int32),
            pltpu.VMEM((2, ech, dw), jnp.int32),
            pltpu.VMEM((2, ech, dw), jnp.int32),
            pltpu.SemaphoreType.DMA,
            pltpu.SemaphoreType.DMA,
            pltpu.SemaphoreType.DMA,
            pltpu.SemaphoreType.DMA,
        ],
        compiler_params=pltpu.CompilerParams(use_tc_tiling_on_sc=False),
    )
    def k(pr_hbm, pm_hbm, src_hbm, dst_hbm, gr_hbm, gm_hbm,
          srcv, dstv, rows_r, rows_m, s0, s1, s2, s3):
        wid = lax.axis_index("s") * _NC + lax.axis_index("c")
        pltpu.sync_copy(src_hbm.at[wid], srcv)
        pltpu.sync_copy(dst_hbm.at[wid], dstv)
        sems = ((s0, s1), (s2, s3))

        def issue(c, b):
            pltpu.async_copy(pr_hbm.at[srcv.at[c]], rows_r.at[b], sems[b][0])
            pltpu.async_copy(pm_hbm.at[dstv.at[c]], rows_m.at[b], sems[b][1])

        issue(0, 0)
        issue(1, 1)

        @pl.loop(0, nch, step=2)
        def chunk(c):
            for b in range(2):
                cc = c + b
                base = pl.multiple_of(wid * epw + cc * ech, 8)
                pltpu.make_async_copy(
                    pr_hbm.at[srcv.at[cc]], rows_r.at[b], sems[b][0]).wait()
                pltpu.sync_copy(rows_r.at[b], gr_hbm.at[pl.ds(base, ech)])
                pltpu.make_async_copy(
                    pm_hbm.at[dstv.at[cc]], rows_m.at[b], sems[b][1]).wait()
                pltpu.sync_copy(rows_m.at[b], gm_hbm.at[pl.ds(base, ech)])

                @pl.when(cc + 2 < nch)
                def _():
                    issue(cc + 2, b)

    return k(pr, pm, src3, dst3)


def _node_aggregate(delta_e, eid, coef, n_mesh):
    npc = 8
    k_deg = eid.shape[2] // npc
    n_pad = eid.shape[0] * eid.shape[1] * npc
    nch = eid.shape[1]
    npw = nch * npc

    @functools.partial(
        pl.kernel,
        out_type=jax.ShapeDtypeStruct((n_pad, D), jnp.float32),
        mesh=_sc_mesh(),
        scratch_types=[
            pltpu.VMEM((nch, npc * k_deg), jnp.int32),
            pltpu.VMEM((nch, npc * k_deg), jnp.float32),
            pltpu.VMEM((2, npc * k_deg, D), jnp.float32),
            pltpu.VMEM((npc, D), jnp.float32),
            pltpu.SemaphoreType.DMA,
            pltpu.SemaphoreType.DMA,
        ],
    )
    def k(de_hbm, eid_hbm, coef_hbm, out_hbm, eidv, coefv, g, outb, s0, s1):
        wid = lax.axis_index("s") * _NC + lax.axis_index("c")
        pltpu.sync_copy(eid_hbm.at[wid], eidv)
        pltpu.sync_copy(coef_hbm.at[wid], coefv)
        sems = (s0, s1)

        def issue(c, b):
            pltpu.async_copy(de_hbm.at[eidv.at[c]], g.at[b], sems[b])

        issue(0, 0)
        issue(1, 1)

        @pl.loop(0, nch, step=2)
        def chunk(c):
            for b in range(2):
                cc = c + b
                pltpu.make_async_copy(
                    de_hbm.at[eidv.at[cc]], g.at[b], sems[b]).wait()

                @plsc.parallel_loop(0, npc, 1, unroll=1)
                def donode(j):
                    acc = [jnp.zeros((_L,), jnp.float32) for _ in range(D // _L)]
                    cj = coefv[cc, pl.ds(j * k_deg, k_deg)]
                    for kk in range(k_deg):
                        row = j * k_deg + kk
                        s = cj[kk]
                        for dj in range(D // _L):
                            acc[dj] = acc[dj] + s * g[b, row, pl.ds(dj * _L, _L)]
                    inv_k = jnp.float32(1.0 / k_deg)
                    for dj in range(D // _L):
                        outb[j, pl.ds(dj * _L, _L)] = acc[dj] * inv_k
                base = pl.multiple_of(wid * npw + cc * npc, 8)
                pltpu.sync_copy(outb, out_hbm.at[pl.ds(base, npc)])

                @pl.when(cc + 2 < nch)
                def _():
                    issue(cc + 2, b)

    return k(delta_e, eid, coef)


def kernel(grid_mesh_bond_embedding, grid_rect_embedding, mesh_node_embedding,
           G2M_edge_id2pair_tensor, G2M_edge_id_of_node_tensor,
           G2M_edge_coef_node_tensor,
           W_GM2E, g_GM2E, b_GM2E, W_E2M, g_E2M, b_E2M, W_G2G, g_G2G, b_G2G):
    bond = grid_mesh_bond_embedding[0]
    rect = grid_rect_embedding[0]
    mesh = mesh_node_embedding[0]
    src = G2M_edge_id2pair_tensor[:, 0]
    dst = G2M_edge_id2pair_tensor[:, 1]

    wb = W_GM2E[:, :D].T
    wr = W_GM2E[:, D:2 * D].T
    wm = W_GM2E[:, 2 * D:].T
    wm1 = W_E2M[:, :D].T
    wm2 = W_E2M[:, D:].T
    wg = W_G2G.T
    g1 = g_GM2E.reshape(1, D)
    b1 = b_GM2E.reshape(1, D)

    pr = _proj_stage(rect, wr, jnp.bfloat16)
    pm = _proj_stage(mesh, wm, jnp.bfloat16)

    gr32, gm32 = _edge_gather(_pack32(pr), _pack32(pm), src, dst)

    delta_e, out_bond = _edge_stage(bond, _unpack16(gr32), _unpack16(gm32),
                                    wb, g1, b1)

    n_mesh = mesh.shape[0]
    k_deg = G2M_edge_id_of_node_tensor.shape[1]
    npc = 8
    npw = -(-n_mesh // _NW)
    npw = -(-npw // (2 * npc)) * (2 * npc)
    n_pad = npw * _NW
    eid_pad = jnp.pad(G2M_edge_id_of_node_tensor, ((0, n_pad - n_mesh), (0, 0)))
    coef_pad = jnp.pad(G2M_edge_coef_node_tensor[..., 0],
                       ((0, n_pad - n_mesh), (0, 0)))
    eid3 = eid_pad.reshape(_NW, npw // npc, npc * k_deg)
    coef3 = coef_pad.reshape(_NW, npw // npc, npc * k_deg)
    agg = _node_aggregate(delta_e, eid3, coef3, n_mesh)[:n_mesh]

    out_mesh = _meshout_stage(mesh, agg, wm1, wm2,
                              g_E2M.reshape(1, D), b_E2M.reshape(1, D))
    out_rect = _resmlp_stage(rect, wg, g_G2G.reshape(1, D), b_G2G.reshape(1, D))

    return (out_bond[None], out_rect[None], out_mesh[None])


# speedup vs baseline: 2.0458x; 2.0458x over previous
import functools

import jax
import jax.numpy as jnp
from jax import lax
from jax.experimental import pallas as pl
from jax.experimental.pallas import tpu as pltpu
from jax.experimental.pallas import tpu_sc as plsc

D = 128
_NC, _NS = 2, 16
_NW = _NC * _NS
_L = 16


def _pick_block(n, candidates=(1280, 1024, 1000, 800, 512, 400, 256, 200, 160, 128, 80, 40, 16, 8)):
    for c in candidates:
        if n % c == 0:
            return c
    return n


def _pack_halves(x):
    u = jax.lax.bitcast_convert_type(x, jnp.int32)
    b = jax.lax.shift_right_logical(
        u + 0x7FFF + (jax.lax.shift_right_logical(u, 16) & 1), 16)
    lo = b[:, :D // 2]
    hi = b[:, D // 2:]
    return jax.lax.shift_left(hi, 16) | lo


def _unpack_halves(w):
    lo = jax.lax.bitcast_convert_type(jax.lax.shift_left(w, 16), jnp.float32)
    hi = jax.lax.bitcast_convert_type(w & jnp.int32(-65536), jnp.float32)
    return jnp.concatenate([lo, hi], axis=-1)


def _ln(x, gamma, beta, eps=1e-5):
    mu = jnp.mean(x, axis=-1, keepdims=True)
    xc = x - mu
    var = jnp.mean(xc * xc, axis=-1, keepdims=True)
    return xc * jax.lax.rsqrt(var + eps) * gamma + beta


def _resmlp_body(x_ref, w_ref, g_ref, b_ref, out_ref):
    x = x_ref[...]
    h = jnp.tanh(jnp.dot(x, w_ref[...], preferred_element_type=jnp.float32))
    out_ref[...] = x + _ln(h, g_ref[...], b_ref[...])


def _proj_body(x_ref, w_ref, out_ref):
    out_ref[...] = _pack_halves(
        jnp.dot(x_ref[...], w_ref[...], preferred_element_type=jnp.float32))


def _edge_body(bond_ref, gr_ref, gm_ref, wb_ref, g_ref, b_ref, de_ref, ob_ref):
    x = bond_ref[...]
    h = jnp.tanh(jnp.dot(x, wb_ref[...], preferred_element_type=jnp.float32)
                 + _unpack_halves(gr_ref[...])
                 + _unpack_halves(gm_ref[...]))
    d = _ln(h, g_ref[...], b_ref[...])
    de_ref[...] = d
    ob_ref[...] = x + d


def _meshout_body(mesh_ref, agg_ref, w1_ref, w2_ref, g_ref, b_ref, out_ref):
    x = mesh_ref[...]
    h = jnp.tanh(jnp.dot(x, w1_ref[...], preferred_element_type=jnp.float32)
                 + jnp.dot(agg_ref[...], w2_ref[...], preferred_element_type=jnp.float32))
    out_ref[...] = x + _ln(h, g_ref[...], b_ref[...])


def _row_spec(tile):
    return pl.BlockSpec((tile, D), lambda i: (i, 0))


def _const_spec(shape):
    return pl.BlockSpec(shape, lambda i: (0,) * len(shape))


def _resmlp_stage(x, w, g, b):
    n = x.shape[0]
    t = _pick_block(n)
    return pl.pallas_call(
        _resmlp_body,
        grid=(n // t,),
        in_specs=[_row_spec(t), _const_spec((D, D)),
                  _const_spec((1, D)), _const_spec((1, D))],
        out_specs=_row_spec(t),
        out_shape=jax.ShapeDtypeStruct((n, D), jnp.float32),
    )(x, w, g, b)


def _proj_stage(x, w):
    n = x.shape[0]
    t = _pick_block(n)
    return pl.pallas_call(
        _proj_body,
        grid=(n // t,),
        in_specs=[_row_spec(t), _const_spec((D, D))],
        out_specs=pl.BlockSpec((t, D // 2), lambda i: (i, 0)),
        out_shape=jax.ShapeDtypeStruct((n, D // 2), jnp.int32),
    )(x, w)


def _edge_stage(bond, gr, gm, wb, g, b):
    n = bond.shape[0]
    t = _pick_block(n)
    half = pl.BlockSpec((t, D // 2), lambda i: (i, 0))
    return pl.pallas_call(
        _edge_body,
        grid=(n // t,),
        in_specs=[_row_spec(t), half, half, _const_spec((D, D)),
                  _const_spec((1, D)), _const_spec((1, D))],
        out_specs=[_row_spec(t), _row_spec(t)],
        out_shape=[jax.ShapeDtypeStruct((n, D), jnp.float32),
                   jax.ShapeDtypeStruct((n, D), jnp.float32)],
    )(bond, gr, gm, wb, g, b)


def _meshout_stage(mesh, agg, w1, w2, g, b):
    n = mesh.shape[0]
    t = _pick_block(n)
    return pl.pallas_call(
        _meshout_body,
        grid=(n // t,),
        in_specs=[_row_spec(t), _row_spec(t), _const_spec((D, D)),
                  _const_spec((D, D)), _const_spec((1, D)), _const_spec((1, D))],
        out_specs=_row_spec(t),
        out_shape=jax.ShapeDtypeStruct((n, D), jnp.float32),
    )(mesh, agg, w1, w2, g, b)


def _sc_mesh():
    return plsc.VectorSubcoreMesh(core_axis_name="c", subcore_axis_name="s",
                                  num_cores=_NC, num_subcores=_NS)


def _edge_gather(pr, pm, src, dst):
    e = src.shape[0]
    ech = 128
    e_pad = -(-e // (_NW * 2 * ech)) * (_NW * 2 * ech)
    epw = e_pad // _NW
    nch = epw // ech
    src3 = jnp.pad(src, (0, e_pad - e)).reshape(_NW, nch, ech)
    dst3 = jnp.pad(dst, (0, e_pad - e)).reshape(_NW, nch, ech)
    dw = pr.shape[1]

    @functools.partial(
        pl.kernel,
        out_type=[jax.ShapeDtypeStruct((e_pad, dw), jnp.int32),
                  jax.ShapeDtypeStruct((e_pad, dw), jnp.int32)],
        mesh=_sc_mesh(),
        scratch_types=[
            pltpu.VMEM((nch, ech), jnp.int32),
            pltpu.VMEM((nch, ech), jnp.int32),
            pltpu.VMEM((2, ech, dw), jnp.int32),
            pltpu.VMEM((2, ech, dw), jnp.int32),
            pltpu.SemaphoreType.DMA,
            pltpu.SemaphoreType.DMA,
            pltpu.SemaphoreType.DMA,
            pltpu.SemaphoreType.DMA,
        ],
        compiler_params=pltpu.CompilerParams(use_tc_tiling_on_sc=False),
    )
    def k(pr_hbm, pm_hbm, src_hbm, dst_hbm, gr_hbm, gm_hbm,
          srcv, dstv, rows_r, rows_m, s0, s1, s2, s3):
        wid = lax.axis_index("s") * _NC + lax.axis_index("c")
        pltpu.sync_copy(src_hbm.at[wid], srcv)
        pltpu.sync_copy(dst_hbm.at[wid], dstv)
        sems = ((s0, s1), (s2, s3))

        def issue(c, b):
            pltpu.async_copy(pr_hbm.at[srcv.at[c]], rows_r.at[b], sems[b][0])
            pltpu.async_copy(pm_hbm.at[dstv.at[c]], rows_m.at[b], sems[b][1])

        issue(0, 0)
        issue(1, 1)

        @pl.loop(0, nch, step=2)
        def chunk(c):
            for b in range(2):
                cc = c + b
                base = pl.multiple_of(wid * epw + cc * ech, 8)
                pltpu.make_async_copy(
                    pr_hbm.at[srcv.at[cc]], rows_r.at[b], sems[b][0]).wait()
                pltpu.sync_copy(rows_r.at[b], gr_hbm.at[pl.ds(base, ech)])
                pltpu.make_async_copy(
                    pm_hbm.at[dstv.at[cc]], rows_m.at[b], sems[b][1]).wait()
                pltpu.sync_copy(rows_m.at[b], gm_hbm.at[pl.ds(base, ech)])

                @pl.when(cc + 2 < nch)
                def _():
                    issue(cc + 2, b)

    return k(pr, pm, src3, dst3)


def _node_aggregate(delta_e, eid, coef, n_mesh):
    npc = 8
    k_deg = eid.shape[2] // npc
    n_pad = eid.shape[0] * eid.shape[1] * npc
    nch = eid.shape[1]
    npw = nch * npc

    @functools.partial(
        pl.kernel,
        out_type=jax.ShapeDtypeStruct((n_pad, D), jnp.float32),
        mesh=_sc_mesh(),
        scratch_types=[
            pltpu.VMEM((nch, npc * k_deg), jnp.int32),
            pltpu.VMEM((nch, npc * k_deg), jnp.float32),
            pltpu.VMEM((2, npc * k_deg, D), jnp.float32),
            pltpu.VMEM((npc, D), jnp.float32),
            pltpu.SemaphoreType.DMA,
            pltpu.SemaphoreType.DMA,
        ],
    )
    def k(de_hbm, eid_hbm, coef_hbm, out_hbm, eidv, coefv, g, outb, s0, s1):
        wid = lax.axis_index("s") * _NC + lax.axis_index("c")
        pltpu.sync_copy(eid_hbm.at[wid], eidv)
        pltpu.sync_copy(coef_hbm.at[wid], coefv)
        sems = (s0, s1)

        def issue(c, b):
            pltpu.async_copy(de_hbm.at[eidv.at[c]], g.at[b], sems[b])

        issue(0, 0)
        issue(1, 1)

        @pl.loop(0, nch, step=2)
        def chunk(c):
            for b in range(2):
                cc = c + b
                pltpu.make_async_copy(
                    de_hbm.at[eidv.at[cc]], g.at[b], sems[b]).wait()

                @plsc.parallel_loop(0, npc, 1, unroll=1)
                def donode(j):
                    acc = [jnp.zeros((_L,), jnp.float32) for _ in range(D // _L)]
                    cj = coefv[cc, pl.ds(j * k_deg, k_deg)]
                    for kk in range(k_deg):
                        row = j * k_deg + kk
                        s = cj[kk]
                        for dj in range(D // _L):
                            acc[dj] = acc[dj] + s * g[b, row, pl.ds(dj * _L, _L)]
                    inv_k = jnp.float32(1.0 / k_deg)
                    for dj in range(D // _L):
                        outb[j, pl.ds(dj * _L, _L)] = acc[dj] * inv_k
                base = pl.multiple_of(wid * npw + cc * npc, 8)
                pltpu.sync_copy(outb, out_hbm.at[pl.ds(base, npc)])

                @pl.when(cc + 2 < nch)
                def _():
                    issue(cc + 2, b)

    return k(delta_e, eid, coef)


def kernel(grid_mesh_bond_embedding, grid_rect_embedding, mesh_node_embedding,
           G2M_edge_id2pair_tensor, G2M_edge_id_of_node_tensor,
           G2M_edge_coef_node_tensor,
           W_GM2E, g_GM2E, b_GM2E, W_E2M, g_E2M, b_E2M, W_G2G, g_G2G, b_G2G):
    bond = grid_mesh_bond_embedding[0]
    rect = grid_rect_embedding[0]
    mesh = mesh_node_embedding[0]
    src = G2M_edge_id2pair_tensor[:, 0]
    dst = G2M_edge_id2pair_tensor[:, 1]

    wb = W_GM2E[:, :D].T
    wr = W_GM2E[:, D:2 * D].T
    wm = W_GM2E[:, 2 * D:].T
    wm1 = W_E2M[:, :D].T
    wm2 = W_E2M[:, D:].T
    wg = W_G2G.T
    g1 = g_GM2E.reshape(1, D)
    b1 = b_GM2E.reshape(1, D)

    pr = _proj_stage(rect, wr)
    pm = _proj_stage(mesh, wm)

    gr32, gm32 = _edge_gather(pr, pm, src, dst)

    delta_e, out_bond = _edge_stage(bond, gr32, gm32, wb, g1, b1)

    n_mesh = mesh.shape[0]
    k_deg = G2M_edge_id_of_node_tensor.shape[1]
    npc = 8
    npw = -(-n_mesh // _NW)
    npw = -(-npw // (2 * npc)) * (2 * npc)
    n_pad = npw * _NW
    eid_pad = jnp.pad(G2M_edge_id_of_node_tensor, ((0, n_pad - n_mesh), (0, 0)))
    coef_pad = jnp.pad(G2M_edge_coef_node_tensor[..., 0],
                       ((0, n_pad - n_mesh), (0, 0)))
    eid3 = eid_pad.reshape(_NW, npw // npc, npc * k_deg)
    coef3 = coef_pad.reshape(_NW, npw // npc, npc * k_deg)
    agg = _node_aggregate(delta_e, eid3, coef3, n_mesh)[:n_mesh]

    out_mesh = _meshout_stage(mesh, agg, wm1, wm2,
                              g_E2M.reshape(1, D), b_E2M.reshape(1, D))
    out_rect = _resmlp_stage(rect, wg, g_G2G.reshape(1, D), b_G2G.reshape(1, D))

    return (out_bond[None], out_rect[None], out_mesh[None])


# speedup vs baseline: 2.0851x; 1.0192x over previous
import functools

import jax
import jax.numpy as jnp
from jax import lax
from jax.experimental import pallas as pl
from jax.experimental.pallas import tpu as pltpu
from jax.experimental.pallas import tpu_sc as plsc

D = 128
_NC, _NS = 2, 16
_NW = _NC * _NS
_L = 16


def _pick_block(n, candidates=(1280, 1024, 1000, 800, 512, 400, 256, 200, 160, 128, 80, 40, 16, 8)):
    for c in candidates:
        if n % c == 0:
            return c
    return n


def _pack_halves(x):
    u = jax.lax.bitcast_convert_type(x, jnp.int32)
    b = jax.lax.shift_right_logical(
        u + 0x7FFF + (jax.lax.shift_right_logical(u, 16) & 1), 16)
    lo = b[:, :D // 2]
    hi = b[:, D // 2:]
    return jax.lax.shift_left(hi, 16) | lo


def _unpack_halves(w):
    lo = jax.lax.bitcast_convert_type(jax.lax.shift_left(w, 16), jnp.float32)
    hi = jax.lax.bitcast_convert_type(w & jnp.int32(-65536), jnp.float32)
    return jnp.concatenate([lo, hi], axis=-1)


def _ln(x, gamma, beta, eps=1e-5):
    mu = jnp.mean(x, axis=-1, keepdims=True)
    xc = x - mu
    var = jnp.mean(xc * xc, axis=-1, keepdims=True)
    return xc * jax.lax.rsqrt(var + eps) * gamma + beta


def _resmlp_body(x_ref, w_ref, g_ref, b_ref, out_ref):
    x = x_ref[...]
    h = jnp.tanh(jnp.dot(x, w_ref[...], preferred_element_type=jnp.float32))
    out_ref[...] = x + _ln(h, g_ref[...], b_ref[...])


def _proj_body(x_ref, w_ref, out_ref):
    out_ref[...] = _pack_halves(
        jnp.dot(x_ref[...], w_ref[...], preferred_element_type=jnp.float32))


def _rect_body(x_ref, wr_ref, wg_ref, g_ref, b_ref, pr_ref, outr_ref):
    x = x_ref[...]
    pr_ref[...] = _pack_halves(
        jnp.dot(x, wr_ref[...], preferred_element_type=jnp.float32))
    h = jnp.tanh(jnp.dot(x, wg_ref[...], preferred_element_type=jnp.float32))
    outr_ref[...] = x + _ln(h, g_ref[...], b_ref[...])


def _edge_body(bond_ref, grgm_ref, wb_ref, g_ref, b_ref, de_ref, ob_ref):
    x = bond_ref[...]
    grgm = grgm_ref[...]
    h = jnp.tanh(jnp.dot(x, wb_ref[...], preferred_element_type=jnp.float32)
                 + _unpack_halves(grgm[:, :D // 2])
                 + _unpack_halves(grgm[:, D // 2:]))
    d = _ln(h, g_ref[...], b_ref[...])
    de_ref[...] = d
    ob_ref[...] = x + d


def _meshout_body(mesh_ref, agg_ref, w1_ref, w2_ref, g_ref, b_ref, out_ref):
    x = mesh_ref[...]
    h = jnp.tanh(jnp.dot(x, w1_ref[...], preferred_element_type=jnp.float32)
                 + jnp.dot(agg_ref[...], w2_ref[...], preferred_element_type=jnp.float32))
    out_ref[...] = x + _ln(h, g_ref[...], b_ref[...])


def _row_spec(tile):
    return pl.BlockSpec((tile, D), lambda i: (i, 0))


def _const_spec(shape):
    return pl.BlockSpec(shape, lambda i: (0,) * len(shape))


def _resmlp_stage(x, w, g, b):
    n = x.shape[0]
    t = _pick_block(n)
    return pl.pallas_call(
        _resmlp_body,
        grid=(n // t,),
        in_specs=[_row_spec(t), _const_spec((D, D)),
                  _const_spec((1, D)), _const_spec((1, D))],
        out_specs=_row_spec(t),
        out_shape=jax.ShapeDtypeStruct((n, D), jnp.float32),
    )(x, w, g, b)


def _proj_stage(x, w):
    n = x.shape[0]
    t = _pick_block(n)
    return pl.pallas_call(
        _proj_body,
        grid=(n // t,),
        in_specs=[_row_spec(t), _const_spec((D, D))],
        out_specs=pl.BlockSpec((t, D // 2), lambda i: (i, 0)),
        out_shape=jax.ShapeDtypeStruct((n, D // 2), jnp.int32),
    )(x, w)


def _edge_stage(bond, grgm, wb, g, b):
    n = bond.shape[0]
    t = _pick_block(n)
    return pl.pallas_call(
        _edge_body,
        grid=(n // t,),
        in_specs=[_row_spec(t), _row_spec(t), _const_spec((D, D)),
                  _const_spec((1, D)), _const_spec((1, D))],
        out_specs=[_row_spec(t), _row_spec(t)],
        out_shape=[jax.ShapeDtypeStruct((n, D), jnp.float32),
                   jax.ShapeDtypeStruct((n, D), jnp.float32)],
    )(bond, grgm, wb, g, b)


def _rect_stage(rect, wr, wg, g, b):
    n = rect.shape[0]
    t = _pick_block(n)
    return pl.pallas_call(
        _rect_body,
        grid=(n // t,),
        in_specs=[_row_spec(t), _const_spec((D, D)), _const_spec((D, D)),
                  _const_spec((1, D)), _const_spec((1, D))],
        out_specs=[pl.BlockSpec((t, D // 2), lambda i: (i, 0)), _row_spec(t)],
        out_shape=[jax.ShapeDtypeStruct((n, D // 2), jnp.int32),
                   jax.ShapeDtypeStruct((n, D), jnp.float32)],
    )(rect, wr, wg, g, b)


def _meshout_stage(mesh, agg, w1, w2, g, b):
    n = mesh.shape[0]
    t = _pick_block(n)
    return pl.pallas_call(
        _meshout_body,
        grid=(n // t,),
        in_specs=[_row_spec(t), _row_spec(t), _const_spec((D, D)),
                  _const_spec((D, D)), _const_spec((1, D)), _const_spec((1, D))],
        out_specs=_row_spec(t),
        out_shape=jax.ShapeDtypeStruct((n, D), jnp.float32),
    )(mesh, agg, w1, w2, g, b)


def _sc_mesh():
    return plsc.VectorSubcoreMesh(core_axis_name="c", subcore_axis_name="s",
                                  num_cores=_NC, num_subcores=_NS)


def _edge_gather(pr, pm, src, dst):
    e = src.shape[0]
    ech = 128
    e_pad = -(-e // (_NW * 2 * ech)) * (_NW * 2 * ech)
    epw = e_pad // _NW
    nch = epw // ech
    src3 = jnp.pad(src, (0, e_pad - e)).reshape(_NW, nch, ech)
    dst3 = jnp.pad(dst, (0, e_pad - e)).reshape(_NW, nch, ech)
    dw = pr.shape[1]

    @functools.partial(
        pl.kernel,
        out_type=jax.ShapeDtypeStruct((e_pad, 2 * dw), jnp.int32),
        mesh=_sc_mesh(),
        scratch_types=[
            pltpu.VMEM((nch, ech), jnp.int32),
            pltpu.VMEM((nch, ech), jnp.int32),
            pltpu.VMEM((2, ech, dw), jnp.int32),
            pltpu.VMEM((2, ech, dw), jnp.int32),
            pltpu.SemaphoreType.DMA,
            pltpu.SemaphoreType.DMA,
            pltpu.SemaphoreType.DMA,
            pltpu.SemaphoreType.DMA,
        ],
        compiler_params=pltpu.CompilerParams(use_tc_tiling_on_sc=False),
    )
    def k(pr_hbm, pm_hbm, src_hbm, dst_hbm, out_hbm,
          srcv, dstv, rows_r, rows_m, s0, s1, s2, s3):
        wid = lax.axis_index("s") * _NC + lax.axis_index("c")
        pltpu.sync_copy(src_hbm.at[wid], srcv)
        pltpu.sync_copy(dst_hbm.at[wid], dstv)
        sems = ((s0, s1), (s2, s3))

        def issue(c, b):
            pltpu.async_copy(pr_hbm.at[srcv.at[c]], rows_r.at[b], sems[b][0])
            pltpu.async_copy(pm_hbm.at[dstv.at[c]], rows_m.at[b], sems[b][1])

        issue(0, 0)
        issue(1, 1)

        @pl.loop(0, nch, step=2)
        def chunk(c):
            for b in range(2):
                cc = c + b
                base = pl.multiple_of(wid * epw + cc * ech, 8)
                pltpu.make_async_copy(
                    pr_hbm.at[srcv.at[cc]], rows_r.at[b], sems[b][0]).wait()
                pltpu.sync_copy(rows_r.at[b],
                                out_hbm.at[pl.ds(base, ech), pl.ds(0, dw)])
                pltpu.make_async_copy(
                    pm_hbm.at[dstv.at[cc]], rows_m.at[b], sems[b][1]).wait()
                pltpu.sync_copy(rows_m.at[b],
                                out_hbm.at[pl.ds(base, ech), pl.ds(dw, dw)])

                @pl.when(cc + 2 < nch)
                def _():
                    issue(cc + 2, b)

    return k(pr, pm, src3, dst3)


def _node_aggregate(delta_e, eid, coef, n_mesh):
    npc = 8
    k_deg = eid.shape[2] // npc
    n_pad = eid.shape[0] * eid.shape[1] * npc
    nch = eid.shape[1]
    npw = nch * npc

    @functools.partial(
        pl.kernel,
        out_type=jax.ShapeDtypeStruct((n_pad, D), jnp.float32),
        mesh=_sc_mesh(),
        scratch_types=[
            pltpu.VMEM((nch, npc * k_deg), jnp.int32),
            pltpu.VMEM((nch, npc * k_deg), jnp.float32),
            pltpu.VMEM((2, npc * k_deg, D), jnp.float32),
            pltpu.VMEM((npc, D), jnp.float32),
            pltpu.SemaphoreType.DMA,
            pltpu.SemaphoreType.DMA,
        ],
    )
    def k(de_hbm, eid_hbm, coef_hbm, out_hbm, eidv, coefv, g, outb, s0, s1):
        wid = lax.axis_index("s") * _NC + lax.axis_index("c")
        pltpu.sync_copy(eid_hbm.at[wid], eidv)
        pltpu.sync_copy(coef_hbm.at[wid], coefv)
        sems = (s0, s1)

        def issue(c, b):
            pltpu.async_copy(de_hbm.at[eidv.at[c]], g.at[b], sems[b])

        issue(0, 0)
        issue(1, 1)

        @pl.loop(0, nch, step=2)
        def chunk(c):
            for b in range(2):
                cc = c + b
                pltpu.make_async_copy(
                    de_hbm.at[eidv.at[cc]], g.at[b], sems[b]).wait()

                @plsc.parallel_loop(0, npc, 1, unroll=1)
                def donode(j):
                    acc = [jnp.zeros((_L,), jnp.float32) for _ in range(D // _L)]
                    cj = coefv[cc, pl.ds(j * k_deg, k_deg)]
                    for kk in range(k_deg):
                        row = j * k_deg + kk
                        s = cj[kk]
                        for dj in range(D // _L):
                            acc[dj] = acc[dj] + s * g[b, row, pl.ds(dj * _L, _L)]
                    inv_k = jnp.float32(1.0 / k_deg)
                    for dj in range(D // _L):
                        outb[j, pl.ds(dj * _L, _L)] = acc[dj] * inv_k
                base = pl.multiple_of(wid * npw + cc * npc, 8)
                pltpu.sync_copy(outb, out_hbm.at[pl.ds(base, npc)])

                @pl.when(cc + 2 < nch)
                def _():
                    issue(cc + 2, b)

    return k(delta_e, eid, coef)


def kernel(grid_mesh_bond_embedding, grid_rect_embedding, mesh_node_embedding,
           G2M_edge_id2pair_tensor, G2M_edge_id_of_node_tensor,
           G2M_edge_coef_node_tensor,
           W_GM2E, g_GM2E, b_GM2E, W_E2M, g_E2M, b_E2M, W_G2G, g_G2G, b_G2G):
    bond = grid_mesh_bond_embedding[0]
    rect = grid_rect_embedding[0]
    mesh = mesh_node_embedding[0]
    src = G2M_edge_id2pair_tensor[:, 0]
    dst = G2M_edge_id2pair_tensor[:, 1]

    wb = W_GM2E[:, :D].T
    wr = W_GM2E[:, D:2 * D].T
    wm = W_GM2E[:, 2 * D:].T
    wm1 = W_E2M[:, :D].T
    wm2 = W_E2M[:, D:].T
    wg = W_G2G.T
    g1 = g_GM2E.reshape(1, D)
    b1 = b_GM2E.reshape(1, D)

    pr, out_rect = _rect_stage(rect, wr, wg,
                               g_G2G.reshape(1, D), b_G2G.reshape(1, D))
    pm = _proj_stage(mesh, wm)

    grgm = _edge_gather(pr, pm, src, dst)

    delta_e, out_bond = _edge_stage(bond, grgm, wb, g1, b1)

    n_mesh = mesh.shape[0]
    k_deg = G2M_edge_id_of_node_tensor.shape[1]
    npc = 8
    npw = -(-n_mesh // _NW)
    npw = -(-npw // (2 * npc)) * (2 * npc)
    n_pad = npw * _NW
    eid_pad = jnp.pad(G2M_edge_id_of_node_tensor, ((0, n_pad - n_mesh), (0, 0)))
    coef_pad = jnp.pad(G2M_edge_coef_node_tensor[..., 0],
                       ((0, n_pad - n_mesh), (0, 0)))
    eid3 = eid_pad.reshape(_NW, npw // npc, npc * k_deg)
    coef3 = coef_pad.reshape(_NW, npw // npc, npc * k_deg)
    agg = _node_aggregate(delta_e, eid3, coef3, n_mesh)[:n_mesh]

    out_mesh = _meshout_stage(mesh, agg, wm1, wm2,
                              g_E2M.reshape(1, D), b_E2M.reshape(1, D))

    return (out_bond[None], out_rect[None], out_mesh[None])


# speedup vs baseline: 2.1186x; 1.0161x over previous
import functools

import jax
import jax.numpy as jnp
from jax import lax
from jax.experimental import pallas as pl
from jax.experimental.pallas import tpu as pltpu
from jax.experimental.pallas import tpu_sc as plsc

D = 128
_NC, _NS = 2, 16
_NW = _NC * _NS
_L = 16


def _pick_block(n, candidates=(1280, 1024, 1000, 800, 512, 400, 256, 200, 160, 128, 80, 40, 16, 8)):
    for c in candidates:
        if n % c == 0:
            return c
    return n


def _pack_halves(x):
    u = jax.lax.bitcast_convert_type(x, jnp.int32)
    b = jax.lax.shift_right_logical(
        u + 0x7FFF + (jax.lax.shift_right_logical(u, 16) & 1), 16)
    lo = b[:, :D // 2]
    hi = b[:, D // 2:]
    return jax.lax.shift_left(hi, 16) | lo


def _unpack_halves(w):
    lo = jax.lax.bitcast_convert_type(jax.lax.shift_left(w, 16), jnp.float32)
    hi = jax.lax.bitcast_convert_type(w & jnp.int32(-65536), jnp.float32)
    return jnp.concatenate([lo, hi], axis=-1)


def _ln(x, gamma, beta, eps=1e-5):
    mu = jnp.mean(x, axis=-1, keepdims=True)
    xc = x - mu
    var = jnp.mean(xc * xc, axis=-1, keepdims=True)
    return xc * jax.lax.rsqrt(var + eps) * gamma + beta


def _resmlp_body(x_ref, w_ref, g_ref, b_ref, out_ref):
    x = x_ref[...]
    h = jnp.tanh(jnp.dot(x, w_ref[...], preferred_element_type=jnp.float32))
    out_ref[...] = x + _ln(h, g_ref[...], b_ref[...])


def _proj_body(x_ref, w_ref, out_ref):
    out_ref[...] = _pack_halves(
        jnp.dot(x_ref[...], w_ref[...], preferred_element_type=jnp.float32))


def _rect_body(x_ref, wr_ref, wg_ref, g_ref, b_ref, pr_ref, outr_ref):
    x = x_ref[...]
    pr_ref[...] = _pack_halves(
        jnp.dot(x, wr_ref[...], preferred_element_type=jnp.float32))
    h = jnp.tanh(jnp.dot(x, wg_ref[...], preferred_element_type=jnp.float32))
    outr_ref[...] = x + _ln(h, g_ref[...], b_ref[...])


def _edge_body(bond_ref, grgm_ref, wb_ref, g_ref, b_ref, de_ref, ob_ref):
    x = bond_ref[...]
    grgm = grgm_ref[...]
    h = jnp.tanh(jnp.dot(x, wb_ref[...], preferred_element_type=jnp.float32)
                 + _unpack_halves(grgm[:, :D // 2])
                 + _unpack_halves(grgm[:, D // 2:]))
    d = _ln(h, g_ref[...], b_ref[...])
    de_ref[...] = d
    ob_ref[...] = x + d


def _meshout_body(mesh_ref, agg_ref, w1_ref, w2_ref, g_ref, b_ref, out_ref):
    x = mesh_ref[...]
    h = jnp.tanh(jnp.dot(x, w1_ref[...], preferred_element_type=jnp.float32)
                 + jnp.dot(agg_ref[...], w2_ref[...], preferred_element_type=jnp.float32))
    out_ref[...] = x + _ln(h, g_ref[...], b_ref[...])


def _row_spec(tile):
    return pl.BlockSpec((tile, D), lambda i: (i, 0))


def _const_spec(shape):
    return pl.BlockSpec(shape, lambda i: (0,) * len(shape))


def _resmlp_stage(x, w, g, b):
    n = x.shape[0]
    t = _pick_block(n)
    return pl.pallas_call(
        _resmlp_body,
        grid=(n // t,),
        in_specs=[_row_spec(t), _const_spec((D, D)),
                  _const_spec((1, D)), _const_spec((1, D))],
        out_specs=_row_spec(t),
        out_shape=jax.ShapeDtypeStruct((n, D), jnp.float32),
    )(x, w, g, b)


def _proj_stage(x, w):
    n = x.shape[0]
    t = _pick_block(n)
    return pl.pallas_call(
        _proj_body,
        grid=(n // t,),
        in_specs=[_row_spec(t), _const_spec((D, D))],
        out_specs=pl.BlockSpec((t, D // 2), lambda i: (i, 0)),
        out_shape=jax.ShapeDtypeStruct((n, D // 2), jnp.int32),
    )(x, w)


def _edge_stage(bond, grgm, wb, g, b):
    n = bond.shape[0]
    t = _pick_block(n)
    return pl.pallas_call(
        _edge_body,
        grid=(n // t,),
        in_specs=[_row_spec(t), _row_spec(t), _const_spec((D, D)),
                  _const_spec((1, D)), _const_spec((1, D))],
        out_specs=[_row_spec(t), _row_spec(t)],
        out_shape=[jax.ShapeDtypeStruct((n, D), jnp.float32),
                   jax.ShapeDtypeStruct((n, D), jnp.float32)],
    )(bond, grgm, wb, g, b)


def _rect_stage(rect, wr, wg, g, b):
    n = rect.shape[0]
    t = _pick_block(n)
    return pl.pallas_call(
        _rect_body,
        grid=(n // t,),
        in_specs=[_row_spec(t), _const_spec((D, D)), _const_spec((D, D)),
                  _const_spec((1, D)), _const_spec((1, D))],
        out_specs=[pl.BlockSpec((t, D // 2), lambda i: (i, 0)), _row_spec(t)],
        out_shape=[jax.ShapeDtypeStruct((n, D // 2), jnp.int32),
                   jax.ShapeDtypeStruct((n, D), jnp.float32)],
    )(rect, wr, wg, g, b)


def _meshout_stage(mesh, agg, w1, w2, g, b):
    n = mesh.shape[0]
    t = _pick_block(n)
    return pl.pallas_call(
        _meshout_body,
        grid=(n // t,),
        in_specs=[_row_spec(t), _row_spec(t), _const_spec((D, D)),
                  _const_spec((D, D)), _const_spec((1, D)), _const_spec((1, D))],
        out_specs=_row_spec(t),
        out_shape=jax.ShapeDtypeStruct((n, D), jnp.float32),
    )(mesh, agg, w1, w2, g, b)


def _sc_mesh():
    return plsc.VectorSubcoreMesh(core_axis_name="c", subcore_axis_name="s",
                                  num_cores=_NC, num_subcores=_NS)


def _edge_gather(pr, pm, src, dst):
    e = src.shape[0]
    ech = 128
    e_pad = -(-e // (_NW * 2 * ech)) * (_NW * 2 * ech)
    epw = e_pad // _NW
    nch = epw // ech
    src3 = jnp.pad(src, (0, e_pad - e)).reshape(_NW, nch, ech)
    dst3 = jnp.pad(dst, (0, e_pad - e)).reshape(_NW, nch, ech)
    dw = pr.shape[1]

    @functools.partial(
        pl.kernel,
        out_type=jax.ShapeDtypeStruct((e_pad, 2 * dw), jnp.int32),
        mesh=_sc_mesh(),
        scratch_types=[
            pltpu.VMEM((nch, ech), jnp.int32),
            pltpu.VMEM((nch, ech), jnp.int32),
            pltpu.VMEM((4, ech, dw), jnp.int32),
            pltpu.VMEM((4, ech, dw), jnp.int32),
            pltpu.SemaphoreType.DMA,
            pltpu.SemaphoreType.DMA,
            pltpu.SemaphoreType.DMA,
            pltpu.SemaphoreType.DMA,
            pltpu.SemaphoreType.DMA,
            pltpu.SemaphoreType.DMA,
            pltpu.SemaphoreType.DMA,
            pltpu.SemaphoreType.DMA,
        ],
        compiler_params=pltpu.CompilerParams(use_tc_tiling_on_sc=False),
    )
    def k(pr_hbm, pm_hbm, src_hbm, dst_hbm, out_hbm,
          srcv, dstv, rows_r, rows_m, s0, s1, s2, s3, s4, s5, s6, s7):
        wid = lax.axis_index("s") * _NC + lax.axis_index("c")
        pltpu.sync_copy(src_hbm.at[wid], srcv)
        pltpu.sync_copy(dst_hbm.at[wid], dstv)
        sems = ((s0, s1), (s2, s3), (s4, s5), (s6, s7))
        nbuf = 4

        def issue(c, b):
            pltpu.async_copy(pr_hbm.at[srcv.at[c]], rows_r.at[b], sems[b][0])
            pltpu.async_copy(pm_hbm.at[dstv.at[c]], rows_m.at[b], sems[b][1])

        for b in range(nbuf):
            issue(b, b)

        @pl.loop(0, nch, step=nbuf)
        def chunk(c):
            for b in range(nbuf):
                cc = c + b
                base = pl.multiple_of(wid * epw + cc * ech, 8)
                pltpu.make_async_copy(
                    pr_hbm.at[srcv.at[cc]], rows_r.at[b], sems[b][0]).wait()
                pltpu.sync_copy(rows_r.at[b],
                                out_hbm.at[pl.ds(base, ech), pl.ds(0, dw)])
                pltpu.make_async_copy(
                    pm_hbm.at[dstv.at[cc]], rows_m.at[b], sems[b][1]).wait()
                pltpu.sync_copy(rows_m.at[b],
                                out_hbm.at[pl.ds(base, ech), pl.ds(dw, dw)])

                @pl.when(cc + nbuf < nch)
                def _():
                    issue(cc + nbuf, b)

    return k(pr, pm, src3, dst3)


def _node_aggregate(delta_e, eid, coef, n_mesh):
    npc = 8
    k_deg = eid.shape[2] // npc
    n_pad = eid.shape[0] * eid.shape[1] * npc
    nch = eid.shape[1]
    npw = nch * npc

    @functools.partial(
        pl.kernel,
        out_type=jax.ShapeDtypeStruct((n_pad, D), jnp.float32),
        mesh=_sc_mesh(),
        scratch_types=[
            pltpu.VMEM((nch, npc * k_deg), jnp.int32),
            pltpu.VMEM((nch, npc * k_deg), jnp.float32),
            pltpu.VMEM((4, npc * k_deg, D), jnp.float32),
            pltpu.VMEM((npc, D), jnp.float32),
            pltpu.SemaphoreType.DMA,
            pltpu.SemaphoreType.DMA,
            pltpu.SemaphoreType.DMA,
            pltpu.SemaphoreType.DMA,
        ],
    )
    def k(de_hbm, eid_hbm, coef_hbm, out_hbm, eidv, coefv, g, outb,
          s0, s1, s2, s3):
        wid = lax.axis_index("s") * _NC + lax.axis_index("c")
        pltpu.sync_copy(eid_hbm.at[wid], eidv)
        pltpu.sync_copy(coef_hbm.at[wid], coefv)
        sems = (s0, s1, s2, s3)
        nbuf = 4

        def issue(c, b):
            pltpu.async_copy(de_hbm.at[eidv.at[c]], g.at[b], sems[b])

        for b in range(nbuf):
            issue(b, b)

        @pl.loop(0, nch, step=nbuf)
        def chunk(c):
            for b in range(nbuf):
                cc = c + b
                pltpu.make_async_copy(
                    de_hbm.at[eidv.at[cc]], g.at[b], sems[b]).wait()

                @plsc.parallel_loop(0, npc, 1, unroll=1)
                def donode(j):
                    acc = [jnp.zeros((_L,), jnp.float32) for _ in range(D // _L)]
                    cj = coefv[cc, pl.ds(j * k_deg, k_deg)]
                    for kk in range(k_deg):
                        row = j * k_deg + kk
                        s = cj[kk]
                        for dj in range(D // _L):
                            acc[dj] = acc[dj] + s * g[b, row, pl.ds(dj * _L, _L)]
                    inv_k = jnp.float32(1.0 / k_deg)
                    for dj in range(D // _L):
                        outb[j, pl.ds(dj * _L, _L)] = acc[dj] * inv_k
                base = pl.multiple_of(wid * npw + cc * npc, 8)
                pltpu.sync_copy(outb, out_hbm.at[pl.ds(base, npc)])

                @pl.when(cc + nbuf < nch)
                def _():
                    issue(cc + nbuf, b)

    return k(delta_e, eid, coef)


def kernel(grid_mesh_bond_embedding, grid_rect_embedding, mesh_node_embedding,
           G2M_edge_id2pair_tensor, G2M_edge_id_of_node_tensor,
           G2M_edge_coef_node_tensor,
           W_GM2E, g_GM2E, b_GM2E, W_E2M, g_E2M, b_E2M, W_G2G, g_G2G, b_G2G):
    bond = grid_mesh_bond_embedding[0]
    rect = grid_rect_embedding[0]
    mesh = mesh_node_embedding[0]
    src = G2M_edge_id2pair_tensor[:, 0]
    dst = G2M_edge_id2pair_tensor[:, 1]

    wb = W_GM2E[:, :D].T
    wr = W_GM2E[:, D:2 * D].T
    wm = W_GM2E[:, 2 * D:].T
    wm1 = W_E2M[:, :D].T
    wm2 = W_E2M[:, D:].T
    wg = W_G2G.T
    g1 = g_GM2E.reshape(1, D)
    b1 = b_GM2E.reshape(1, D)

    pr, out_rect = _rect_stage(rect, wr, wg,
                               g_G2G.reshape(1, D), b_G2G.reshape(1, D))
    pm = _proj_stage(mesh, wm)

    grgm = _edge_gather(pr, pm, src, dst)

    delta_e, out_bond = _edge_stage(bond, grgm, wb, g1, b1)

    n_mesh = mesh.shape[0]
    k_deg = G2M_edge_id_of_node_tensor.shape[1]
    npc = 8
    npw = -(-n_mesh // _NW)
    npw = -(-npw // (2 * npc)) * (2 * npc)
    n_pad = npw * _NW
    eid_pad = jnp.pad(G2M_edge_id_of_node_tensor, ((0, n_pad - n_mesh), (0, 0)))
    coef_pad = jnp.pad(G2M_edge_coef_node_tensor[..., 0],
                       ((0, n_pad - n_mesh), (0, 0)))
    eid3 = eid_pad.reshape(_NW, npw // npc, npc * k_deg)
    coef3 = coef_pad.reshape(_NW, npw // npc, npc * k_deg)
    agg = _node_aggregate(delta_e, eid3, coef3, n_mesh)[:n_mesh]

    out_mesh = _meshout_stage(mesh, agg, wm1, wm2,
                              g_E2M.reshape(1, D), b_E2M.reshape(1, D))

    return (out_bond[None], out_rect[None], out_mesh[None])
